# baseline (speedup 1.0000x reference)
import functools

import jax
import jax.numpy as jnp
import numpy as np
from jax import lax
from jax.experimental import pallas as pl
from jax.experimental.pallas import tpu as pltpu
from jax.experimental.pallas import tpu_sc as plsc

MAX_NGRAM = 3
N_HEADS = 8
EMBED_DIM = 32
N_EMBD = 1024
TABLE_BASE = 200000
K_TAPS = 4
DILATION = 3
B, T = 4, 4096
BT = B * T
D_MEM = (MAX_NGRAM - 1) * N_HEADS * EMBED_DIM

PADW = 4112
IDSV = 528


def _is_prime(m):
    return all(m % d for d in range(3, int(m ** 0.5) + 1, 2))


def _hash_specs():
    specs = []
    for n in range(2, MAX_NGRAM + 1):
        for k in range(N_HEADS):
            m = TABLE_BASE + n * k * 997
            if m > 2 and m % 2 == 0:
                m += 1
            while m > 2 and not _is_prime(m):
                m += 2
            prime = 2 if m <= 2 else m
            seed = n * 2654435761 + k * 40503
            powers = [pow(seed, n - j, prime) for j in range(n)]
            specs.append((n, prime, powers))
    return specs


_SPECS = _hash_specs()


def _sc_gather(ids_pad_flat, tables):
    info = plsc.get_sparse_core_info()
    NC, NS = info.num_cores, info.num_subcores
    NW = NC * NS
    R = BT // NW
    NVEC = R // 16

    def body(ids_hbm, *rest):
        tbls = rest[:16]
        out_hbm = rest[16]
        ids_v, idx_v, rows_v, sem = rest[17:]
        i32 = jnp.int32
        wid = lax.axis_index("s") * i32(NC) + lax.axis_index("c")
        base = wid * i32(R)
        bb = base // i32(T)
        t0 = base % i32(T)
        pltpu.sync_copy(ids_hbm.at[pl.ds(bb * i32(PADW) + t0, IDSV)], ids_v)

        for ti, (n, prime, powers) in enumerate(_SPECS):
            inv_p = jnp.float32(1.0 / prime)
            p_i = jnp.int32(prime)

            def mulmod(w, c):
                hi, lo = int(c) >> 4, int(c) & 15
                a = w * jnp.int32(hi)
                q = (a.astype(jnp.float32) * inv_p).astype(jnp.int32)
                r = a - q * p_i
                r = jnp.where(r < 0, r + p_i, r)
                r = jnp.where(r >= p_i, r - p_i, r)
                return r * jnp.int32(16) + w * jnp.int32(lo)

            def compute(i, carry):
                l = i * i32(16)
                h = mulmod(ids_v[pl.ds(l + i32(3 - n), 16)], powers[0])
                h = h + mulmod(ids_v[pl.ds(l + i32(4 - n), 16)], powers[1])
                if n == 3:
                    h = h + mulmod(ids_v[pl.ds(l + i32(2), 16)], powers[2])
                q = (h.astype(jnp.float32) * inv_p).astype(jnp.int32)
                r = h - q * p_i
                r = jnp.where(r < 0, r + p_i, r)
                r = jnp.where(r >= p_i, r - p_i, r)
                idx_v[pl.ds(l, 16)] = r
                return carry

            lax.fori_loop(jnp.int32(0), jnp.int32(NVEC), compute,
                          jnp.int32(0))

            copies = [
                pltpu.async_copy(
                    tbls[ti].at[idx_v.at[pl.ds(j * 128, 128)]],
                    rows_v.at[pl.ds(j * 128, 128)],
                    sem,
                )
                for j in range(R // 128)
            ]
            for cp in copies:
                cp.wait()
            pltpu.sync_copy(
                rows_v,
                out_hbm.at[pl.ds(base, R), pl.ds(ti * EMBED_DIM, EMBED_DIM)],
            )

    kfn = pl.kernel(
        body,
        out_type=jax.ShapeDtypeStruct((BT, D_MEM), jnp.float32),
        mesh=plsc.VectorSubcoreMesh(core_axis_name="c", subcore_axis_name="s"),
        scratch_types=[
            pltpu.VMEM((IDSV,), jnp.int32),
            pltpu.VMEM((R,), jnp.int32),
            pltpu.VMEM((R, EMBED_DIM), jnp.float32),
            pltpu.SemaphoreType.DMA,
        ],
        compiler_params=pltpu.CompilerParams(use_tc_tiling_on_sc=False),
    )
    return kfn(ids_pad_flat, *tables)


CT = 1024
CPB = T // CT
_Z = np.int32(0)


def _dense_body(x_ref, e_ref, wk_ref, wv_ref, wc_ref, lo_ref, hi_ref,
                hist_ref):
    c = pl.program_id(0)

    @pl.when(c % CPB == 0)
    def _():
        hist_ref[...] = jnp.zeros_like(hist_ref)

    eps = jnp.float32(1.1920929e-07)
    x = x_ref[...]
    e = e_ref[...]
    kkr = lax.dot_general(e, wk_ref[...], (((1,), (1,)), ((), ())),
                          preferred_element_type=jnp.float32)
    v = lax.dot_general(e, wv_ref[...], (((1,), (1,)), ((), ())),
                        preferred_element_type=jnp.float32)
    kk = kkr * lax.rsqrt(jnp.mean(kkr * kkr, axis=-1, keepdims=True) + eps)
    xn = x * lax.rsqrt(jnp.mean(x * x, axis=-1, keepdims=True) + eps)
    gate = jax.nn.sigmoid(
        jnp.sum(xn * kk, axis=-1, keepdims=True) * jnp.float32(1.0 / 32.0))
    gv = gate * v
    nv = gv * lax.rsqrt(jnp.mean(gv * gv, axis=-1, keepdims=True) + eps)
    ext = jnp.concatenate([hist_ref[...], nv], axis=0)
    wc = wc_ref[...]
    conv = ext[7:7 + CT] * wc[0:1]
    for j in range(1, K_TAPS):
        o = 7 + j * DILATION
        conv = conv + ext[o:o + CT] * wc[j:j + 1]
    y = conv * jax.nn.sigmoid(conv) + gv
    hist_ref[...] = nv[CT - 16:]

    u = jnp.uint32
    bits = lax.bitcast_convert_type(y, jnp.uint32)
    ex = (bits >> u(23)) & u(0xFF)
    m = bits & u(0x7FFFFF)
    e64 = jnp.where(ex == u(255), u(2047), ex + u(896))
    sub = ex == u(0)
    hi_ref[...] = (bits & u(0x80000000)) | jnp.where(
        sub, u(0), (e64 << u(20)) | (m >> u(3)))
    lo_ref[...] = jnp.where(sub, u(0), m << u(29))


def _dense(x2, e, Wk, Wv, WconvT, interpret=False):
    return pl.pallas_call(
        _dense_body,
        grid=(BT // CT,),
        in_specs=[
            pl.BlockSpec((CT, N_EMBD), lambda i: (i, _Z)),
            pl.BlockSpec((CT, D_MEM), lambda i: (i, _Z)),
            pl.BlockSpec((N_EMBD, D_MEM), lambda i: (_Z, _Z)),
            pl.BlockSpec((N_EMBD, D_MEM), lambda i: (_Z, _Z)),
            pl.BlockSpec((K_TAPS, N_EMBD), lambda i: (_Z, _Z)),
        ],
        out_specs=[pl.BlockSpec((CT, N_EMBD), lambda i: (i, _Z)),
                   pl.BlockSpec((CT, N_EMBD), lambda i: (i, _Z))],
        out_shape=[jax.ShapeDtypeStruct((BT, N_EMBD), jnp.uint32),
                   jax.ShapeDtypeStruct((BT, N_EMBD), jnp.uint32)],
        scratch_shapes=[pltpu.VMEM((16, N_EMBD), jnp.float32)],
        compiler_params=pltpu.CompilerParams(
            vmem_limit_bytes=100 * 1024 * 1024),
        interpret=interpret,
    )(x2, e, Wk, Wv, WconvT)


def kernel(x, input_ids, Wk, Wv, Wconv, t_0, t_1, t_2, t_3, t_4, t_5, t_6,
           t_7, t_8, t_9, t_10, t_11, t_12, t_13, t_14, t_15):
    tables = [t_0, t_1, t_2, t_3, t_4, t_5, t_6, t_7,
              t_8, t_9, t_10, t_11, t_12, t_13, t_14, t_15]
    ids32 = input_ids.astype(jnp.int32)
    ids_pad = jnp.zeros((B, PADW), jnp.int32).at[:, 2:2 + T].set(ids32)
    e = _sc_gather(ids_pad.reshape(-1), tables)
    lo, hi = _dense(x.reshape(BT, N_EMBD), e, Wk.astype(jnp.float32),
                    Wv.astype(jnp.float32), Wconv.T.astype(jnp.float32))
    st = jnp.stack([lo, hi], axis=-1)
    out = lax.bitcast_convert_type(st, jnp.float64)
    return out.reshape(B, T, N_EMBD)


# speedup vs baseline: 1.1058x; 1.1058x over previous
import jax
import jax.numpy as jnp
import numpy as np
from jax import lax
from jax.experimental import pallas as pl
from jax.experimental.pallas import tpu as pltpu
from jax.experimental.pallas import tpu_sc as plsc

MAX_NGRAM = 3
N_HEADS = 8
EMBED_DIM = 32
N_EMBD = 1024
TABLE_BASE = 200000
K_TAPS = 4
DILATION = 3
B, T = 4, 4096
BT = B * T
D_MEM = (MAX_NGRAM - 1) * N_HEADS * EMBED_DIM

PADW = 4112
IDSV = 528


def _is_prime(m):
    return all(m % d for d in range(3, int(m ** 0.5) + 1, 2))


def _hash_specs():
    specs = []
    for n in range(2, MAX_NGRAM + 1):
        for k in range(N_HEADS):
            m = TABLE_BASE + n * k * 997
            if m > 2 and m % 2 == 0:
                m += 1
            while m > 2 and not _is_prime(m):
                m += 2
            prime = 2 if m <= 2 else m
            seed = n * 2654435761 + k * 40503
            powers = [pow(seed, n - j, prime) for j in range(n)]
            specs.append((n, prime, powers))
    return specs


_SPECS = _hash_specs()


def _sc_gather(ids_pad_flat, tables):
    info = plsc.get_sparse_core_info()
    NC, NS = info.num_cores, info.num_subcores
    NW = NC * NS
    R = BT // NW
    NVEC = R // 16

    def body(ids_hbm, *rest):
        tbls = rest[:16]
        out_hbm = rest[16]
        ids_v, idx_v, rows_v, sem = rest[17:]
        i32 = jnp.int32
        wid = lax.axis_index("s") * i32(NC) + lax.axis_index("c")
        base = wid * i32(R)
        bb = base // i32(T)
        t0 = base % i32(T)
        pltpu.sync_copy(ids_hbm.at[pl.ds(bb * i32(PADW) + t0, IDSV)], ids_v)

        for ti, (n, prime, powers) in enumerate(_SPECS):
            inv_p = jnp.float32(1.0 / prime)
            p_i = jnp.int32(prime)

            def mulmod(w, c):
                hi, lo = int(c) >> 4, int(c) & 15
                a = w * jnp.int32(hi)
                q = (a.astype(jnp.float32) * inv_p).astype(jnp.int32)
                r = a - q * p_i
                r = jnp.where(r < 0, r + p_i, r)
                r = jnp.where(r >= p_i, r - p_i, r)
                return r * jnp.int32(16) + w * jnp.int32(lo)

            def compute(i, carry):
                l = i * i32(16)
                h = mulmod(ids_v[pl.ds(l + i32(3 - n), 16)], powers[0])
                h = h + mulmod(ids_v[pl.ds(l + i32(4 - n), 16)], powers[1])
                if n == 3:
                    h = h + mulmod(ids_v[pl.ds(l + i32(2), 16)], powers[2])
                q = (h.astype(jnp.float32) * inv_p).astype(jnp.int32)
                r = h - q * p_i
                r = jnp.where(r < 0, r + p_i, r)
                r = jnp.where(r >= p_i, r - p_i, r)
                idx_v[pl.ds(l, 16)] = r
                return carry

            lax.fori_loop(jnp.int32(0), jnp.int32(NVEC), compute,
                          jnp.int32(0))

            copies = [
                pltpu.async_copy(
                    tbls[ti].at[idx_v.at[pl.ds(j * 128, 128)]],
                    rows_v.at[pl.ds(j * 128, 128)],
                    sem,
                )
                for j in range(R // 128)
            ]
            for cp in copies:
                cp.wait()
            pltpu.sync_copy(
                rows_v,
                out_hbm.at[pl.ds(base, R), pl.ds(ti * EMBED_DIM, EMBED_DIM)],
            )

    kfn = pl.kernel(
        body,
        out_type=jax.ShapeDtypeStruct((BT, D_MEM), jnp.float32),
        mesh=plsc.VectorSubcoreMesh(core_axis_name="c", subcore_axis_name="s"),
        scratch_types=[
            pltpu.VMEM((IDSV,), jnp.int32),
            pltpu.VMEM((R,), jnp.int32),
            pltpu.VMEM((R, EMBED_DIM), jnp.float32),
            pltpu.SemaphoreType.DMA,
        ],
        compiler_params=pltpu.CompilerParams(use_tc_tiling_on_sc=False),
    )
    return kfn(ids_pad_flat, *tables)


CT = 1024
CPB = T // CT
_Z = np.int32(0)


def _dense_body(x_ref, e_ref, wk_ref, wv_ref, wc_ref, out_ref, hist_ref):
    c = pl.program_id(0)

    @pl.when(c % CPB == 0)
    def _():
        hist_ref[...] = jnp.zeros_like(hist_ref)

    eps = jnp.float32(1.1920929e-07)
    x = x_ref[...]
    e = e_ref[...]
    kkr = lax.dot_general(e, wk_ref[...], (((1,), (1,)), ((), ())),
                          preferred_element_type=jnp.float32)
    v = lax.dot_general(e, wv_ref[...], (((1,), (1,)), ((), ())),
                        preferred_element_type=jnp.float32)
    kk = kkr * lax.rsqrt(jnp.mean(kkr * kkr, axis=-1, keepdims=True) + eps)
    xn = x * lax.rsqrt(jnp.mean(x * x, axis=-1, keepdims=True) + eps)
    gate = jax.nn.sigmoid(
        jnp.sum(xn * kk, axis=-1, keepdims=True) * jnp.float32(1.0 / 32.0))
    gv = gate * v
    nv = gv * lax.rsqrt(jnp.mean(gv * gv, axis=-1, keepdims=True) + eps)
    ext = jnp.concatenate([hist_ref[...], nv], axis=0)
    wc = wc_ref[...]
    conv = ext[7:7 + CT] * wc[0:1]
    for j in range(1, K_TAPS):
        o = 7 + j * DILATION
        conv = conv + ext[o:o + CT] * wc[j:j + 1]
    out_ref[...] = conv * jax.nn.sigmoid(conv) + gv
    hist_ref[...] = nv[CT - 16:]


def _dense(x2, e, Wk, Wv, WconvT):
    return pl.pallas_call(
        _dense_body,
        grid=(BT // CT,),
        in_specs=[
            pl.BlockSpec((CT, N_EMBD), lambda i: (i, _Z)),
            pl.BlockSpec((CT, D_MEM), lambda i: (i, _Z)),
            pl.BlockSpec((N_EMBD, D_MEM), lambda i: (_Z, _Z)),
            pl.BlockSpec((N_EMBD, D_MEM), lambda i: (_Z, _Z)),
            pl.BlockSpec((K_TAPS, N_EMBD), lambda i: (_Z, _Z)),
        ],
        out_specs=pl.BlockSpec((CT, N_EMBD), lambda i: (i, _Z)),
        out_shape=jax.ShapeDtypeStruct((BT, N_EMBD), jnp.float32),
        scratch_shapes=[pltpu.VMEM((16, N_EMBD), jnp.float32)],
    )(x2, e, Wk, Wv, WconvT)


def kernel(x, input_ids, Wk, Wv, Wconv, t_0, t_1, t_2, t_3, t_4, t_5, t_6,
           t_7, t_8, t_9, t_10, t_11, t_12, t_13, t_14, t_15):
    tables = [t_0, t_1, t_2, t_3, t_4, t_5, t_6, t_7,
              t_8, t_9, t_10, t_11, t_12, t_13, t_14, t_15]
    ids32 = input_ids.astype(jnp.int32)
    ids_pad = jnp.zeros((B, PADW), jnp.int32).at[:, 2:2 + T].set(ids32)
    e = _sc_gather(ids_pad.reshape(-1), tables)
    out = _dense(x.reshape(BT, N_EMBD), e, Wk.astype(jnp.float32),
                 Wv.astype(jnp.float32), Wconv.T.astype(jnp.float32))
    return out.reshape(B, T, N_EMBD).astype(jnp.float64)


# speedup vs baseline: 1.1063x; 1.0004x over previous
import jax
import jax.numpy as jnp
import numpy as np
from jax import lax
from jax.experimental import pallas as pl
from jax.experimental.pallas import tpu as pltpu
from jax.experimental.pallas import tpu_sc as plsc

MAX_NGRAM = 3
N_HEADS = 8
EMBED_DIM = 32
N_EMBD = 1024
TABLE_BASE = 200000
K_TAPS = 4
DILATION = 3
B, T = 4, 4096
BT = B * T
D_MEM = (MAX_NGRAM - 1) * N_HEADS * EMBED_DIM

PADW = 4112
IDSV = 528


def _is_prime(m):
    return all(m % d for d in range(3, int(m ** 0.5) + 1, 2))


def _hash_specs():
    specs = []
    for n in range(2, MAX_NGRAM + 1):
        for k in range(N_HEADS):
            m = TABLE_BASE + n * k * 997
            if m > 2 and m % 2 == 0:
                m += 1
            while m > 2 and not _is_prime(m):
                m += 2
            prime = 2 if m <= 2 else m
            seed = n * 2654435761 + k * 40503
            powers = [pow(seed, n - j, prime) for j in range(n)]
            specs.append((n, prime, powers))
    return specs


_SPECS = _hash_specs()


def _sc_gather(ids_pad_flat, tables):
    info = plsc.get_sparse_core_info()
    NC, NS = info.num_cores, info.num_subcores
    NW = NC * NS
    R = BT // NW
    NVEC = R // 16

    def body(ids_hbm, *rest):
        tbls = rest[:16]
        out_hbm = rest[16]
        ids_v, idx_v, rows_v, sem = rest[17:]
        i32 = jnp.int32
        wid = lax.axis_index("s") * i32(NC) + lax.axis_index("c")
        base = wid * i32(R)
        bb = base // i32(T)
        t0 = base % i32(T)
        pltpu.sync_copy(ids_hbm.at[pl.ds(bb * i32(PADW) + t0, IDSV)], ids_v)

        for ti, (n, prime, powers) in enumerate(_SPECS):
            inv_p = jnp.float32(1.0 / prime)
            p_i = jnp.int32(prime)

            def mulmod(w, c):
                hi, lo = int(c) >> 4, int(c) & 15
                a = w * jnp.int32(hi)
                q = (a.astype(jnp.float32) * inv_p).astype(jnp.int32)
                r = a - q * p_i
                r = jnp.where(r < 0, r + p_i, r)
                r = jnp.where(r >= p_i, r - p_i, r)
                return r * jnp.int32(16) + w * jnp.int32(lo)

            def compute(i, carry):
                l = i * i32(16)
                h = mulmod(ids_v[pl.ds(l + i32(3 - n), 16)], powers[0])
                h = h + mulmod(ids_v[pl.ds(l + i32(4 - n), 16)], powers[1])
                if n == 3:
                    h = h + mulmod(ids_v[pl.ds(l + i32(2), 16)], powers[2])
                q = (h.astype(jnp.float32) * inv_p).astype(jnp.int32)
                r = h - q * p_i
                r = jnp.where(r < 0, r + p_i, r)
                r = jnp.where(r >= p_i, r - p_i, r)
                idx_v[pl.ds(l, 16)] = r
                return carry

            lax.fori_loop(jnp.int32(0), jnp.int32(NVEC), compute,
                          jnp.int32(0))

            copies = [
                pltpu.async_copy(
                    tbls[ti].at[idx_v.at[pl.ds(j * 128, 128)]],
                    rows_v.at[pl.ds(j * 128, 128)],
                    sem,
                )
                for j in range(R // 128)
            ]
            for cp in copies:
                cp.wait()
            pltpu.sync_copy(
                rows_v,
                out_hbm.at[pl.ds(base, R), pl.ds(ti * EMBED_DIM, EMBED_DIM)],
            )

    kfn = pl.kernel(
        body,
        out_type=jax.ShapeDtypeStruct((BT, D_MEM), jnp.float32),
        mesh=plsc.VectorSubcoreMesh(core_axis_name="c", subcore_axis_name="s"),
        scratch_types=[
            pltpu.VMEM((IDSV,), jnp.int32),
            pltpu.VMEM((R,), jnp.int32),
            pltpu.VMEM((R, EMBED_DIM), jnp.float32),
            pltpu.SemaphoreType.DMA,
        ],
        compiler_params=pltpu.CompilerParams(use_tc_tiling_on_sc=False),
    )
    return kfn(ids_pad_flat, *tables)


CT = 1024
CPB = T // CT
_Z = np.int32(0)


def _dense_body(x_ref, e_ref, wk_ref, wv_ref, wc_ref, out_ref, hist_ref):
    c = pl.program_id(0)

    @pl.when(c % CPB == 0)
    def _():
        hist_ref[...] = jnp.zeros_like(hist_ref)

    eps = jnp.float32(1.1920929e-07)
    x = x_ref[...]
    eb = e_ref[...].astype(jnp.bfloat16)
    kkr = lax.dot_general(eb, wk_ref[...].astype(jnp.bfloat16),
                          (((1,), (1,)), ((), ())),
                          preferred_element_type=jnp.float32)
    v = lax.dot_general(eb, wv_ref[...].astype(jnp.bfloat16),
                        (((1,), (1,)), ((), ())),
                        preferred_element_type=jnp.float32)
    kk = kkr * lax.rsqrt(jnp.mean(kkr * kkr, axis=-1, keepdims=True) + eps)
    xn = x * lax.rsqrt(jnp.mean(x * x, axis=-1, keepdims=True) + eps)
    gate = jax.nn.sigmoid(
        jnp.sum(xn * kk, axis=-1, keepdims=True) * jnp.float32(1.0 / 32.0))
    gv = gate * v
    nv = gv * lax.rsqrt(jnp.mean(gv * gv, axis=-1, keepdims=True) + eps)
    ext = jnp.concatenate([hist_ref[...], nv], axis=0)
    wc = wc_ref[...]
    conv = ext[7:7 + CT] * wc[0:1]
    for j in range(1, K_TAPS):
        o = 7 + j * DILATION
        conv = conv + ext[o:o + CT] * wc[j:j + 1]
    out_ref[...] = conv * jax.nn.sigmoid(conv) + gv
    hist_ref[...] = nv[CT - 16:]


def _dense(x2, e, Wk, Wv, WconvT):
    return pl.pallas_call(
        _dense_body,
        grid=(BT // CT,),
        in_specs=[
            pl.BlockSpec((CT, N_EMBD), lambda i: (i, _Z)),
            pl.BlockSpec((CT, D_MEM), lambda i: (i, _Z)),
            pl.BlockSpec((N_EMBD, D_MEM), lambda i: (_Z, _Z)),
            pl.BlockSpec((N_EMBD, D_MEM), lambda i: (_Z, _Z)),
            pl.BlockSpec((K_TAPS, N_EMBD), lambda i: (_Z, _Z)),
        ],
        out_specs=pl.BlockSpec((CT, N_EMBD), lambda i: (i, _Z)),
        out_shape=jax.ShapeDtypeStruct((BT, N_EMBD), jnp.float32),
        scratch_shapes=[pltpu.VMEM((16, N_EMBD), jnp.float32)],
    )(x2, e, Wk, Wv, WconvT)


def kernel(x, input_ids, Wk, Wv, Wconv, t_0, t_1, t_2, t_3, t_4, t_5, t_6,
           t_7, t_8, t_9, t_10, t_11, t_12, t_13, t_14, t_15):
    tables = [t_0, t_1, t_2, t_3, t_4, t_5, t_6, t_7,
              t_8, t_9, t_10, t_11, t_12, t_13, t_14, t_15]
    ids32 = input_ids.astype(jnp.int32)
    ids_pad = jnp.zeros((B, PADW), jnp.int32).at[:, 2:2 + T].set(ids32)
    e = _sc_gather(ids_pad.reshape(-1), tables)
    out = _dense(x.reshape(BT, N_EMBD), e, Wk.astype(jnp.float32),
                 Wv.astype(jnp.float32), Wconv.T.astype(jnp.float32))
    return out.reshape(B, T, N_EMBD).astype(jnp.float64)
